# baseline (speedup 1.0000x reference)
import functools

import jax
import jax.numpy as jnp
from jax import lax
from jax.experimental import pallas as pl
from jax.experimental.pallas import tpu as pltpu
from jax.experimental.pallas import tpu_sc as plsc

VOCAB = 100000
EMBED = 32
BATCH = 16384
SEQ = 200

NC, NS = 2, 16
NW = NC * NS
NBT = 4
NITEMS = SEQ * NBT
RG = 16
RW = 4
RT = 8
UNROLL = 32
NOUTER = NITEMS // UNROLL


@functools.partial(
    pl.kernel,
    out_type=jax.ShapeDtypeStruct((SEQ, EMBED // 8, BATCH // 128, 8, 128),
                                  jnp.float32),
    mesh=plsc.VectorSubcoreMesh(core_axis_name="c", subcore_axis_name="s"),
    scratch_types=[
        pltpu.VMEM((RT, NBT, 128), jnp.int32),
        pltpu.VMEM((RG, 128, EMBED), jnp.float32),
        pltpu.VMEM((RW, EMBED // 8, 8, 128), jnp.float32),
        pltpu.VMEM((128, EMBED + 1), jnp.float32),
        [pltpu.SemaphoreType.DMA] * RT,
        [pltpu.SemaphoreType.DMA] * RG,
        [pltpu.SemaphoreType.DMA] * RW,
    ],
    compiler_params=pltpu.CompilerParams(use_tc_tiling_on_sc=False,
                                         needs_layout_passes=False),
)
def _embed_all(tokt_hbm, tbl_hbm, out_hbm, tok_v, grows_v, tbuf_v, pad_v,
               sem_t, sem_g, sem_o):
    wid = lax.axis_index("s") * NC + lax.axis_index("c")
    b0w = wid * NBT * 128
    bt0 = wid * NBT

    lanes = lax.iota(jnp.int32, 16)
    lane_blocks = [lanes + (l * 16) for l in range(8)]

    def fire_tok(sq, slot):
        for j in range(NBT):
            pltpu.async_copy(tokt_hbm.at[sq, pl.ds(b0w + j * 128, 128)],
                             tok_v.at[slot, j], sem_t[slot])

    def drain_tok(sq, slot):
        for j in range(NBT):
            pltpu.make_async_copy(tokt_hbm.at[sq, pl.ds(b0w + j * 128, 128)],
                                  tok_v.at[slot, j], sem_t[slot]).wait()

    def fire_gather(sq, tslot, j, rg):
        pltpu.async_copy(tbl_hbm.at[tok_v.at[tslot, j]],
                         grows_v.at[rg], sem_g[rg])

    def drain_gather(rg):
        pltpu.make_async_copy(tbl_hbm.at[pl.ds(0, 128)], grows_v.at[rg],
                              sem_g[rg]).wait()

    def fire_write(sq, j, rw):
        pltpu.async_copy(tbuf_v.at[rw],
                         out_hbm.at[sq, pl.ds(0, EMBED // 8), bt0 + j],
                         sem_o[rw])

    def drain_write(sq, j, rw):
        pltpu.make_async_copy(tbuf_v.at[rw],
                              out_hbm.at[sq, pl.ds(0, EMBED // 8), bt0 + j],
                              sem_o[rw]).wait()

    def transpose(rg, rw):
        def c_body(c, carry):
            for u in range(4):
                bl = c * 4 + u
                v0 = grows_v[rg, bl, pl.ds(0, 16)]
                v1 = grows_v[rg, bl, pl.ds(16, 16)]
                pad_v[bl, pl.ds(0, 16)] = v0
                pad_v[bl, pl.ds(16, 16)] = v1
            return carry

        lax.fori_loop(0, 32, c_body, 0)

        def m_body(m, carry):
            et = m // 8
            er = m % 8
            col_v = jnp.full((16,), m, jnp.int32)
            vs = [plsc.load_gather(pad_v, [lane_blocks[l], col_v])
                  for l in range(8)]
            for l in range(8):
                tbuf_v[rw, et, er, pl.ds(l * 16, 16)] = vs[l]
            return carry

        lax.fori_loop(0, EMBED, m_body, 0)

    for sq in range(4):
        fire_tok(sq, sq)
    for sq in range(4):
        drain_tok(sq, sq)
    for i in range(RG):
        fire_gather(i // NBT, i // NBT, i % NBT, i)
    for sq in range(4, 8):
        fire_tok(sq, sq)

    def outer(p, carry):
        s0 = p * (UNROLL // NBT)
        for o in range(UNROLL):
            rg = o % RG
            rw = o % RW
            j = o % NBT
            q = o // NBT
            sq = s0 + q

            drain_gather(rg)

            def _drain_prev():
                drain_write(sq - 1, j, rw)
            if o < RW:
                pl.when(p >= 1)(_drain_prev)
            else:
                _drain_prev()

            transpose(rg, rw)
            fire_write(sq, j, rw)

            if o % NBT == 0:
                pl.when(s0 + 4 + q <= SEQ - 1)(
                    lambda: drain_tok(s0 + 4 + q, (4 + q) % RT))
            if o % NBT == 3:
                pl.when(s0 + 8 + q <= SEQ - 1)(
                    lambda: fire_tok(s0 + 8 + q, q % RT))

            def _fg():
                fire_gather(s0 + (o + RG) // NBT,
                            ((o + RG) // NBT) % RT, j, rg)
            if o + RG < UNROLL:
                _fg()
            else:
                pl.when(p < NOUTER - 1)(_fg)

        return carry

    lax.fori_loop(0, NOUTER, outer, 0)

    for t in range(RW):
        drain_write(SEQ - 1, t, t)


def kernel(inchi, table, start_var):
    tokt = jnp.pad(inchi.astype(jnp.int32).T, ((1, 0), (0, 0)),
                   constant_values=VOCAB)[:SEQ]
    tbl = jnp.concatenate([table, start_var], axis=0)
    out5 = _embed_all(tokt, tbl)
    return out5.transpose((2, 4, 0, 1, 3)).reshape(BATCH, SEQ, EMBED)
